# Initial kernel scaffold; baseline (speedup 1.0000x reference)
import jax
import jax.numpy as jnp
from jax.experimental import pallas as pl


def kernel(x, hour_w, weekday_w, day_w, month_w):
    raise NotImplementedError("write your pallas kernel here")


# speedup vs baseline: 7.9966x; 7.9966x over previous
import jax
import jax.numpy as jnp
from jax.experimental import pallas as pl

D_MODEL = 2048
ROWS = 32768
R_BLK = 256
K_PAD = 128
OFF_H, OFF_W, OFF_D, OFF_M = 0, 24, 31, 63


def _embed_block(idx_ref, tcat_ref, out_ref):
    idx = idx_ref[...]
    j = jax.lax.broadcasted_iota(jnp.int32, (R_BLK, K_PAD), 1)
    hit = (
        (j == OFF_H + idx[:, 3:4])
        | (j == OFF_W + idx[:, 2:3])
        | (j == OFF_D + idx[:, 1:2])
        | (j == OFF_M + idx[:, 0:1])
    )
    onehot = hit.astype(jnp.float32)
    out_ref[...] = jnp.dot(onehot, tcat_ref[...],
                           preferred_element_type=jnp.float32)


def kernel(x, hour_w, weekday_w, day_w, month_w):
    b, s, _ = x.shape
    x2 = x.reshape(ROWS, 4).astype(jnp.int32)
    tcat = jnp.concatenate([hour_w, weekday_w, day_w, month_w], axis=0)
    tcat = jnp.pad(tcat, ((0, K_PAD - tcat.shape[0]), (0, 0)))
    out = pl.pallas_call(
        _embed_block,
        grid=(ROWS // R_BLK,),
        in_specs=[
            pl.BlockSpec((R_BLK, 4), lambda i: (i, 0)),
            pl.BlockSpec((K_PAD, D_MODEL), lambda i: (0, 0)),
        ],
        out_specs=pl.BlockSpec((R_BLK, D_MODEL), lambda i: (i, 0)),
        out_shape=jax.ShapeDtypeStruct((ROWS, D_MODEL), jnp.float32),
    )(x2, tcat)
    return out.reshape(b, s, D_MODEL)
